# baseline (speedup 1.0000x reference)
import functools

import jax
import jax.numpy as jnp
from jax import lax
from jax.experimental import pallas as pl
from jax.experimental.pallas import tpu as pltpu
from jax.experimental.pallas import tpu_sc as plsc

N = 10000
E = 320000
D_IN = 128
D_HID = 16
D_OUT = 128

NC = 2
NS = 16
NW = NC * NS
SUB = 128
NCHUNK = E // SUB
CPW = NCHUNK // NW
TAIL = NCHUNK - CPW * NW
NBUF = 13
NP = 10240
NPF = NP * D_HID // 128
ROWS_PER_SUB = NP // NS
WPS = NP // NS

_sc_mesh = plsc.VectorSubcoreMesh(core_axis_name="c", subcore_axis_name="s")


def _fill_indices(idx_hbm, wid, idx_v, isem, with_tail):
    base = wid * CPW * SUB

    def fill(t, carry):
        pltpu.async_copy(
            idx_hbm.at[pl.ds(base + t * SUB, SUB)], idx_v.at[t], isem
        )
        return carry

    lax.fori_loop(0, CPW, fill, 0)
    if with_tail:
        @pl.when(wid < TAIL)
        def _():
            pltpu.async_copy(
                idx_hbm.at[pl.ds((NW * CPW + wid) * SUB, SUB)],
                idx_v.at[CPW],
                isem,
            )

    def drain(t, carry):
        pltpu.make_async_copy(
            idx_hbm.at[pl.ds(base + t * SUB, SUB)], idx_v.at[t], isem
        ).wait()
        return carry

    lax.fori_loop(0, CPW, drain, 0)
    if with_tail:
        @pl.when(wid < TAIL)
        def _():
            pltpu.make_async_copy(
                idx_hbm.at[pl.ds((NW * CPW + wid) * SUB, SUB)],
                idx_v.at[CPW],
                isem,
            ).wait()


@functools.partial(
    pl.kernel,
    out_type=jax.ShapeDtypeStruct((NC, NP), jnp.float32),
    mesh=_sc_mesh,
    compiler_params=pltpu.CompilerParams(use_tc_tiling_on_sc=False),
    scratch_types=[
        pltpu.VMEM((CPW + 1, SUB), jnp.int32),
        pltpu.VMEM((SUB,), jnp.float32),
        pltpu.SemaphoreType.DMA,
        pltpu.SemaphoreType.DMA,
        pltpu.VMEM_SHARED((NP,), jnp.float32),
    ],
)
def _deg(dst_hbm, zeros_hbm, out_hbm, dst_v, ones_v, isem, dsem, acc):
    cid = lax.axis_index("c")
    sid = lax.axis_index("s")
    wid = sid * NC + cid
    w0 = sid * WPS
    pltpu.sync_copy(zeros_hbm.at[pl.ds(0, WPS)], acc.at[pl.ds(w0, WPS)])
    _fill_indices(dst_hbm, wid, dst_v, isem, with_tail=True)
    for i in range(SUB // 16):
        ones_v[pl.ds(i * 16, 16)] = jnp.full((16,), 1.0, jnp.float32)
    plsc.subcore_barrier()

    def fire(j, carry):
        pltpu.async_copy(ones_v, acc.at[dst_v.at[j]], dsem, add=True)
        return carry

    lax.fori_loop(0, CPW, fire, 0)

    @pl.when(wid < TAIL)
    def _():
        pltpu.async_copy(ones_v, acc.at[dst_v.at[CPW]], dsem, add=True)

    def drain(j, carry):
        pltpu.make_async_copy(ones_v, acc.at[dst_v.at[j]], dsem).wait()
        return carry

    lax.fori_loop(0, CPW, drain, 0)

    @pl.when(wid < TAIL)
    def _():
        pltpu.make_async_copy(ones_v, acc.at[dst_v.at[CPW]], dsem).wait()

    plsc.subcore_barrier()
    pltpu.sync_copy(acc.at[pl.ds(w0, WPS)], out_hbm.at[cid, pl.ds(w0, WPS)])


@functools.partial(
    pl.kernel,
    out_type=jax.ShapeDtypeStruct((NC, NP, D_HID), jnp.float32),
    mesh=_sc_mesh,
    compiler_params=pltpu.CompilerParams(use_tc_tiling_on_sc=False),
    scratch_types=[
        pltpu.VMEM((CPW + 1, SUB), jnp.int32),
        pltpu.VMEM((CPW + 1, SUB), jnp.int32),
        pltpu.VMEM((NBUF, SUB, D_HID), jnp.float32),
        pltpu.SemaphoreType.DMA,
        pltpu.SemaphoreType.DMA((NBUF,)),
        pltpu.VMEM_SHARED((NP, D_HID), jnp.float32),
    ],
)
def _agg(y_hbm, src_hbm, dst_hbm, zeros_hbm, out_hbm, src_v, dst_v, rows_v, isem, sem, acc):
    cid = lax.axis_index("c")
    sid = lax.axis_index("s")
    wid = sid * NC + cid
    r0 = sid * ROWS_PER_SUB
    pltpu.sync_copy(zeros_hbm.at[pl.ds(r0, ROWS_PER_SUB)], acc.at[pl.ds(r0, ROWS_PER_SUB)])
    _fill_indices(src_hbm, wid, src_v, isem, with_tail=True)
    _fill_indices(dst_hbm, wid, dst_v, isem, with_tail=True)
    plsc.subcore_barrier()

    for b in range(NBUF):
        pltpu.async_copy(y_hbm.at[src_v.at[b]], rows_v.at[b], sem.at[b])

    def outer(o, carry):
        for b in range(NBUF):
            j = o * NBUF + b
            pltpu.make_async_copy(
                y_hbm.at[src_v.at[j]], rows_v.at[b], sem.at[b]
            ).wait()
            pltpu.sync_copy(rows_v.at[b], acc.at[dst_v.at[j]], add=True)
            nxt = j + NBUF

            @pl.when(nxt < CPW)
            def _():
                pltpu.async_copy(y_hbm.at[src_v.at[nxt]], rows_v.at[b], sem.at[b])

        return carry

    lax.fori_loop(0, CPW // NBUF, outer, 0)

    @pl.when(wid < TAIL)
    def _():
        pltpu.async_copy(y_hbm.at[src_v.at[CPW]], rows_v.at[0], sem.at[0])
        pltpu.make_async_copy(y_hbm.at[src_v.at[CPW]], rows_v.at[0], sem.at[0]).wait()
        pltpu.sync_copy(rows_v.at[0], acc.at[dst_v.at[CPW]], add=True)

    plsc.subcore_barrier()
    pltpu.sync_copy(acc.at[pl.ds(r0, ROWS_PER_SUB)], out_hbm.at[cid, pl.ds(r0, ROWS_PER_SUB)])


def _scale_body(degp_ref, dinv_ref):
    deg = degp_ref[0] + degp_ref[1] + 1.0
    dinv_ref[...] = lax.rsqrt(deg)


def _scale(degp):
    return pl.pallas_call(
        _scale_body,
        out_shape=jax.ShapeDtypeStruct((NP // 128, 128), jnp.float32),
    )(degp)


def _mm1_body(xf_ref, wb_ref, dinv_ref, o_ref):
    xw = jnp.dot(xf_ref[...], wb_ref[...], preferred_element_type=jnp.float32)
    o_ref[...] = dinv_ref[...] * xw


def _mm1(xf, wb, dinv16f):
    return pl.pallas_call(
        _mm1_body,
        out_shape=jax.ShapeDtypeStruct((NPF, 128), jnp.float32),
    )(xf, wb, dinv16f)


def _mid_body(sp_ref, y1_ref, dinv_ref, b_ref, o_ref):
    agg = dinv_ref[...] * (sp_ref[0] + sp_ref[1] + y1_ref[...])
    h = jnp.maximum(agg + b_ref[...], 0.0)
    o_ref[...] = dinv_ref[...] * h


def _mid(sp, y1, dinv16, b1t):
    return pl.pallas_call(
        _mid_body,
        out_shape=jax.ShapeDtypeStruct((NPF, 128), jnp.float32),
    )(sp, y1, dinv16, b1t)


def _final_body(sp_ref, y2_ref, dinv_ref, wb_ref, b_ref, o_ref):
    zf = dinv_ref[...] * (sp_ref[0] + sp_ref[1] + y2_ref[...])
    o_ref[...] = (
        jnp.dot(zf, wb_ref[...], preferred_element_type=jnp.float32) + b_ref[...]
    )


def _final(sp, y2, dinv16, w2b, b2t):
    return pl.pallas_call(
        _final_body,
        out_shape=jax.ShapeDtypeStruct((NPF, 8 * D_OUT), jnp.float32),
    )(sp, y2, dinv16, w2b, b2t)


def kernel(data, edge_index, W1, b1, W2, b2):
    f32 = jnp.float32
    zerosf = jnp.zeros((NPF, 128), f32)
    b1t = jnp.tile(b1, 128 // D_HID).reshape(1, 128)
    w1b = (jnp.eye(8, dtype=f32)[:, None, :, None] * W1[None, :, None, :]).reshape(
        8 * D_IN, 8 * D_HID
    )
    xf = jnp.concatenate([data, jnp.zeros((NP - N, D_IN), f32)], axis=0).reshape(
        NP // 8, 8 * D_IN
    )

    src1d = edge_index[0]
    dst1d = edge_index[1]
    degp = _deg(dst1d, zerosf.reshape(NPF * 128))
    dinvf = _scale(degp.reshape(NC, NP // 128, 128))
    dinv16f = jnp.broadcast_to(
        dinvf.reshape(NP, 1), (NP, D_HID)
    ).reshape(NPF, 128)
    y1f = _mm1(xf, w1b, dinv16f)
    s1p = _agg(y1f.reshape(NP, D_HID), src1d, dst1d, zerosf.reshape(NP, D_HID))
    y2f = _mid(s1p.reshape(NC, NPF, 128), y1f, dinv16f, b1t)
    s2p = _agg(y2f.reshape(NP, D_HID), src1d, dst1d, zerosf.reshape(NP, D_HID))
    w2b = (jnp.eye(8, dtype=f32)[:, None, :, None] * W2[None, :, None, :]).reshape(
        8 * D_HID, 8 * D_OUT
    )
    b2t = jnp.tile(b2, 8).reshape(1, 8 * D_OUT)
    outf = _final(s2p.reshape(NC, NPF, 128), y2f, dinv16f, w2b, b2t)
    return outf.reshape(NP, D_OUT)[:N]


# speedup vs baseline: 1.0528x; 1.0528x over previous
import functools

import jax
import jax.numpy as jnp
from jax import lax
from jax.experimental import pallas as pl
from jax.experimental.pallas import tpu as pltpu
from jax.experimental.pallas import tpu_sc as plsc

N = 10000
E = 320000
D_IN = 128
D_HID = 16
D_OUT = 128

NC = 2
NS = 16
NW = NC * NS
SUB = 128
NCHUNK = E // SUB
CPW = NCHUNK // NW
TAIL = NCHUNK - CPW * NW
NBUF = 13
NP = 10240
NPF = NP * D_HID // 128
ROWS_PER_SUB = NP // NS
WPS = NP // NS

_sc_mesh = plsc.VectorSubcoreMesh(core_axis_name="c", subcore_axis_name="s")


def _fill_indices(edge_hbm, row, wid, idx_v, isem, with_tail):
    base = wid * CPW * SUB

    def fill(t, carry):
        pltpu.async_copy(
            edge_hbm.at[row, pl.ds(base + t * SUB, SUB)], idx_v.at[t], isem
        )
        return carry

    lax.fori_loop(0, CPW, fill, 0)
    if with_tail:
        @pl.when(wid < TAIL)
        def _():
            pltpu.async_copy(
                edge_hbm.at[row, pl.ds((NW * CPW + wid) * SUB, SUB)],
                idx_v.at[CPW],
                isem,
            )

    def drain(t, carry):
        pltpu.make_async_copy(
            edge_hbm.at[row, pl.ds(base + t * SUB, SUB)], idx_v.at[t], isem
        ).wait()
        return carry

    lax.fori_loop(0, CPW, drain, 0)
    if with_tail:
        @pl.when(wid < TAIL)
        def _():
            pltpu.make_async_copy(
                edge_hbm.at[row, pl.ds((NW * CPW + wid) * SUB, SUB)],
                idx_v.at[CPW],
                isem,
            ).wait()


@functools.partial(
    pl.kernel,
    out_type=jax.ShapeDtypeStruct((NC, NP), jnp.float32),
    mesh=_sc_mesh,
    compiler_params=pltpu.CompilerParams(use_tc_tiling_on_sc=False),
    scratch_types=[
        pltpu.VMEM((CPW + 1, SUB), jnp.int32),
        pltpu.VMEM((SUB,), jnp.float32),
        pltpu.SemaphoreType.DMA,
        pltpu.SemaphoreType.DMA,
        pltpu.VMEM_SHARED((NP,), jnp.float32),
    ],
)
def _deg(edge_hbm, zeros_hbm, out_hbm, dst_v, ones_v, isem, dsem, acc):
    cid = lax.axis_index("c")
    sid = lax.axis_index("s")
    wid = sid * NC + cid
    w0 = sid * WPS
    pltpu.sync_copy(zeros_hbm.at[pl.ds(0, WPS)], acc.at[pl.ds(w0, WPS)])
    _fill_indices(edge_hbm, 1, wid, dst_v, isem, with_tail=True)
    for i in range(SUB // 16):
        ones_v[pl.ds(i * 16, 16)] = jnp.full((16,), 1.0, jnp.float32)
    plsc.subcore_barrier()

    def fire(j, carry):
        pltpu.async_copy(ones_v, acc.at[dst_v.at[j]], dsem, add=True)
        return carry

    lax.fori_loop(0, CPW, fire, 0)

    @pl.when(wid < TAIL)
    def _():
        pltpu.async_copy(ones_v, acc.at[dst_v.at[CPW]], dsem, add=True)

    def drain(j, carry):
        pltpu.make_async_copy(ones_v, acc.at[dst_v.at[j]], dsem).wait()
        return carry

    lax.fori_loop(0, CPW, drain, 0)

    @pl.when(wid < TAIL)
    def _():
        pltpu.make_async_copy(ones_v, acc.at[dst_v.at[CPW]], dsem).wait()

    plsc.subcore_barrier()
    pltpu.sync_copy(acc.at[pl.ds(w0, WPS)], out_hbm.at[cid, pl.ds(w0, WPS)])


@functools.partial(
    pl.kernel,
    out_type=jax.ShapeDtypeStruct((NC, NP, D_HID), jnp.float32),
    mesh=_sc_mesh,
    compiler_params=pltpu.CompilerParams(use_tc_tiling_on_sc=False),
    scratch_types=[
        pltpu.VMEM((CPW + 1, SUB), jnp.int32),
        pltpu.VMEM((CPW + 1, SUB), jnp.int32),
        pltpu.VMEM((NBUF, SUB, D_HID), jnp.float32),
        pltpu.SemaphoreType.DMA,
        pltpu.SemaphoreType.DMA((NBUF,)),
        pltpu.VMEM_SHARED((NP, D_HID), jnp.float32),
        pltpu.VMEM_SHARED((NP, D_HID), jnp.float32),
    ],
)
def _agg(y_hbm, edge_hbm, zeros_hbm, out_hbm, src_v, dst_v, rows_v, isem, sem, acc, ytab):
    cid = lax.axis_index("c")
    sid = lax.axis_index("s")
    wid = sid * NC + cid
    r0 = sid * ROWS_PER_SUB
    pltpu.sync_copy(zeros_hbm.at[pl.ds(r0, ROWS_PER_SUB)], acc.at[pl.ds(r0, ROWS_PER_SUB)])
    pltpu.sync_copy(y_hbm.at[pl.ds(r0, ROWS_PER_SUB)], ytab.at[pl.ds(r0, ROWS_PER_SUB)])
    _fill_indices(edge_hbm, 0, wid, src_v, isem, with_tail=True)
    _fill_indices(edge_hbm, 1, wid, dst_v, isem, with_tail=True)
    plsc.subcore_barrier()

    for b in range(NBUF):
        pltpu.async_copy(ytab.at[src_v.at[b]], rows_v.at[b], sem.at[b])

    def outer(o, carry):
        for b in range(NBUF):
            j = o * NBUF + b
            pltpu.make_async_copy(
                ytab.at[src_v.at[j]], rows_v.at[b], sem.at[b]
            ).wait()
            pltpu.sync_copy(rows_v.at[b], acc.at[dst_v.at[j]], add=True)
            nxt = j + NBUF

            @pl.when(nxt < CPW)
            def _():
                pltpu.async_copy(ytab.at[src_v.at[nxt]], rows_v.at[b], sem.at[b])

        return carry

    lax.fori_loop(0, CPW // NBUF, outer, 0)

    @pl.when(wid < TAIL)
    def _():
        pltpu.async_copy(ytab.at[src_v.at[CPW]], rows_v.at[0], sem.at[0])
        pltpu.make_async_copy(ytab.at[src_v.at[CPW]], rows_v.at[0], sem.at[0]).wait()
        pltpu.sync_copy(rows_v.at[0], acc.at[dst_v.at[CPW]], add=True)

    plsc.subcore_barrier()
    pltpu.sync_copy(acc.at[pl.ds(r0, ROWS_PER_SUB)], out_hbm.at[cid, pl.ds(r0, ROWS_PER_SUB)])


def _scale_body(degp_ref, dinv_ref):
    deg = degp_ref[0] + degp_ref[1] + 1.0
    dinv_ref[...] = lax.rsqrt(deg)


def _scale(degp):
    return pl.pallas_call(
        _scale_body,
        out_shape=jax.ShapeDtypeStruct((NP // 128, 128), jnp.float32),
    )(degp)


def _mm1_body(xf_ref, wb_ref, dinv_ref, o_ref):
    xw = jnp.dot(xf_ref[...], wb_ref[...], preferred_element_type=jnp.float32)
    o_ref[...] = dinv_ref[...] * xw


def _mm1(xf, wb, dinv16f):
    return pl.pallas_call(
        _mm1_body,
        out_shape=jax.ShapeDtypeStruct((NPF, 128), jnp.float32),
    )(xf, wb, dinv16f)


def _mid_body(sp_ref, y1_ref, dinv_ref, b_ref, o_ref):
    agg = dinv_ref[...] * (sp_ref[0] + sp_ref[1] + y1_ref[...])
    h = jnp.maximum(agg + b_ref[...], 0.0)
    o_ref[...] = dinv_ref[...] * h


def _mid(sp, y1, dinv16, b1t):
    return pl.pallas_call(
        _mid_body,
        out_shape=jax.ShapeDtypeStruct((NPF, 128), jnp.float32),
    )(sp, y1, dinv16, b1t)


def _final_body(sp_ref, y2_ref, dinv_ref, wb_ref, b_ref, o_ref):
    zf = dinv_ref[...] * (sp_ref[0] + sp_ref[1] + y2_ref[...])
    o_ref[...] = (
        jnp.dot(zf, wb_ref[...], preferred_element_type=jnp.float32) + b_ref[...]
    )


def _final(sp, y2, dinv16, w2b, b2t):
    return pl.pallas_call(
        _final_body,
        out_shape=jax.ShapeDtypeStruct((NPF, 8 * D_OUT), jnp.float32),
    )(sp, y2, dinv16, w2b, b2t)


def kernel(data, edge_index, W1, b1, W2, b2):
    f32 = jnp.float32
    zerosf = jnp.zeros((NPF, 128), f32)
    b1t = jnp.tile(b1, 128 // D_HID).reshape(1, 128)
    w1b = (jnp.eye(8, dtype=f32)[:, None, :, None] * W1[None, :, None, :]).reshape(
        8 * D_IN, 8 * D_HID
    )
    xf = jnp.concatenate([data, jnp.zeros((NP - N, D_IN), f32)], axis=0).reshape(
        NP // 8, 8 * D_IN
    )

    degp = _deg(edge_index, zerosf.reshape(NPF * 128))
    dinvf = _scale(degp.reshape(NC, NP // 128, 128))
    dinv16f = jnp.broadcast_to(
        dinvf.reshape(NP, 1), (NP, D_HID)
    ).reshape(NPF, 128)
    y1f = _mm1(xf, w1b, dinv16f)
    s1p = _agg(y1f.reshape(NP, D_HID), edge_index, zerosf.reshape(NP, D_HID))
    y2f = _mid(s1p.reshape(NC, NPF, 128), y1f, dinv16f, b1t)
    s2p = _agg(y2f.reshape(NP, D_HID), edge_index, zerosf.reshape(NP, D_HID))
    w2b = (jnp.eye(8, dtype=f32)[:, None, :, None] * W2[None, :, None, :]).reshape(
        8 * D_HID, 8 * D_OUT
    )
    b2t = jnp.tile(b2, 8).reshape(1, 8 * D_OUT)
    outf = _final(s2p.reshape(NC, NPF, 128), y2f, dinv16f, w2b, b2t)
    return outf.reshape(NP, D_OUT)[:N]


# speedup vs baseline: 1.1846x; 1.1252x over previous
import functools

import jax
import jax.numpy as jnp
from jax import lax
from jax.experimental import pallas as pl
from jax.experimental.pallas import tpu as pltpu
from jax.experimental.pallas import tpu_sc as plsc

N = 10000
E = 320000
D_IN = 128
D_HID = 16
D_OUT = 128

NC = 2
NS = 16
NW = NC * NS
SUB = 128
NCHUNK = E // SUB
CPW = NCHUNK // NW
TAIL = NCHUNK - CPW * NW
NBUF = 13
NP = 10240
NPF = NP * D_HID // 128
ROWS_PER_SUB = NP // NS
WPS = NP // NS

_sc_mesh = plsc.VectorSubcoreMesh(core_axis_name="c", subcore_axis_name="s")


def _fill_indices(edge_hbm, row, wid, idx_v, isem, with_tail):
    base = wid * CPW * SUB

    def fill(t, carry):
        pltpu.async_copy(
            edge_hbm.at[row, pl.ds(base + t * SUB, SUB)], idx_v.at[t], isem
        )
        return carry

    lax.fori_loop(0, CPW, fill, 0)
    if with_tail:
        @pl.when(wid < TAIL)
        def _():
            pltpu.async_copy(
                edge_hbm.at[row, pl.ds((NW * CPW + wid) * SUB, SUB)],
                idx_v.at[CPW],
                isem,
            )

    def drain(t, carry):
        pltpu.make_async_copy(
            edge_hbm.at[row, pl.ds(base + t * SUB, SUB)], idx_v.at[t], isem
        ).wait()
        return carry

    lax.fori_loop(0, CPW, drain, 0)
    if with_tail:
        @pl.when(wid < TAIL)
        def _():
            pltpu.make_async_copy(
                edge_hbm.at[row, pl.ds((NW * CPW + wid) * SUB, SUB)],
                idx_v.at[CPW],
                isem,
            ).wait()


@functools.partial(
    pl.kernel,
    out_type=jax.ShapeDtypeStruct((NC, NP), jnp.float32),
    mesh=_sc_mesh,
    compiler_params=pltpu.CompilerParams(use_tc_tiling_on_sc=False),
    scratch_types=[
        pltpu.VMEM((CPW + 1, SUB), jnp.int32),
        pltpu.VMEM((SUB,), jnp.float32),
        pltpu.SemaphoreType.DMA,
        pltpu.SemaphoreType.DMA,
        pltpu.VMEM_SHARED((NP,), jnp.float32),
    ],
)
def _deg(edge_hbm, zeros_hbm, out_hbm, dst_v, ones_v, isem, dsem, acc):
    cid = lax.axis_index("c")
    sid = lax.axis_index("s")
    wid = sid * NC + cid
    w0 = sid * WPS
    pltpu.sync_copy(zeros_hbm.at[pl.ds(0, WPS)], acc.at[pl.ds(w0, WPS)])
    _fill_indices(edge_hbm, 1, wid, dst_v, isem, with_tail=True)
    for i in range(SUB // 16):
        ones_v[pl.ds(i * 16, 16)] = jnp.full((16,), 1.0, jnp.float32)
    plsc.subcore_barrier()

    def fire(j, carry):
        pltpu.async_copy(ones_v, acc.at[dst_v.at[j]], dsem, add=True)
        return carry

    lax.fori_loop(0, CPW, fire, 0)

    @pl.when(wid < TAIL)
    def _():
        pltpu.async_copy(ones_v, acc.at[dst_v.at[CPW]], dsem, add=True)

    def drain(j, carry):
        pltpu.make_async_copy(ones_v, acc.at[dst_v.at[j]], dsem).wait()
        return carry

    lax.fori_loop(0, CPW, drain, 0)

    @pl.when(wid < TAIL)
    def _():
        pltpu.make_async_copy(ones_v, acc.at[dst_v.at[CPW]], dsem).wait()

    plsc.subcore_barrier()
    pltpu.sync_copy(acc.at[pl.ds(w0, WPS)], out_hbm.at[cid, pl.ds(w0, WPS)])


@functools.partial(
    pl.kernel,
    out_type=jax.ShapeDtypeStruct((NC, NP, D_HID), jnp.float32),
    mesh=_sc_mesh,
    compiler_params=pltpu.CompilerParams(use_tc_tiling_on_sc=False),
    scratch_types=[
        pltpu.VMEM((CPW + 1, SUB), jnp.int32),
        pltpu.VMEM((CPW + 1, SUB), jnp.int32),
        pltpu.VMEM((NBUF, SUB, D_HID), jnp.float32),
        pltpu.SemaphoreType.DMA,
        pltpu.SemaphoreType.DMA((NBUF,)),
        pltpu.VMEM_SHARED((NP, D_HID), jnp.float32),
    ],
)
def _agg(y_hbm, edge_hbm, zeros_hbm, out_hbm, src_v, dst_v, rows_v, isem, sem, acc):
    cid = lax.axis_index("c")
    sid = lax.axis_index("s")
    wid = sid * NC + cid
    r0 = sid * ROWS_PER_SUB
    pltpu.sync_copy(zeros_hbm.at[pl.ds(r0, ROWS_PER_SUB)], acc.at[pl.ds(r0, ROWS_PER_SUB)])
    _fill_indices(edge_hbm, 0, wid, src_v, isem, with_tail=True)
    _fill_indices(edge_hbm, 1, wid, dst_v, isem, with_tail=True)
    plsc.subcore_barrier()

    for b in range(NBUF):
        pltpu.async_copy(y_hbm.at[src_v.at[b]], rows_v.at[b], sem.at[b])

    def outer(o, carry):
        for b in range(NBUF):
            j = o * NBUF + b
            pltpu.make_async_copy(
                y_hbm.at[src_v.at[j]], rows_v.at[b], sem.at[b]
            ).wait()
            pltpu.sync_copy(rows_v.at[b], acc.at[dst_v.at[j]], add=True)
            nxt = j + NBUF

            @pl.when(nxt < CPW)
            def _():
                pltpu.async_copy(y_hbm.at[src_v.at[nxt]], rows_v.at[b], sem.at[b])

        return carry

    lax.fori_loop(0, CPW // NBUF, outer, 0)

    @pl.when(wid < TAIL)
    def _():
        pltpu.async_copy(y_hbm.at[src_v.at[CPW]], rows_v.at[0], sem.at[0])
        pltpu.make_async_copy(y_hbm.at[src_v.at[CPW]], rows_v.at[0], sem.at[0]).wait()
        pltpu.sync_copy(rows_v.at[0], acc.at[dst_v.at[CPW]], add=True)

    plsc.subcore_barrier()
    pltpu.sync_copy(acc.at[pl.ds(r0, ROWS_PER_SUB)], out_hbm.at[cid, pl.ds(r0, ROWS_PER_SUB)])


def _scale_body(degp_ref, dinv_ref):
    deg = degp_ref[0] + degp_ref[1] + 1.0
    dinv_ref[...] = lax.rsqrt(deg)


def _scale(degp):
    return pl.pallas_call(
        _scale_body,
        out_shape=jax.ShapeDtypeStruct((NP // 128, 128), jnp.float32),
    )(degp)


def _mm1_body(x_hbm, wb_ref, dinv_ref, o_ref, xf_ref, dsem):
    pltpu.async_copy(x_hbm.reshape(N // 8, 8 * D_IN), xf_ref, dsem).wait()
    xw = jnp.dot(xf_ref[...], wb_ref[...], preferred_element_type=jnp.float32)
    o_ref[: N // 8] = dinv_ref[: N // 8] * xw
    o_ref[N // 8 :] = jnp.zeros((NPF - N // 8, 128), jnp.float32)


def _mm1(x, wb, dinv16f):
    return pl.pallas_call(
        _mm1_body,
        in_specs=[
            pl.BlockSpec(memory_space=pltpu.MemorySpace.HBM),
            pl.BlockSpec(memory_space=pltpu.MemorySpace.VMEM),
            pl.BlockSpec(memory_space=pltpu.MemorySpace.VMEM),
        ],
        scratch_shapes=[
            pltpu.VMEM((N // 8, 8 * D_IN), jnp.float32),
            pltpu.SemaphoreType.DMA,
        ],
        out_shape=jax.ShapeDtypeStruct((NPF, 128), jnp.float32),
    )(x, wb, dinv16f)


def _mid_body(sp_ref, y1_ref, dinv_ref, b_ref, o_ref):
    agg = dinv_ref[...] * (sp_ref[0] + sp_ref[1] + y1_ref[...])
    h = jnp.maximum(agg + b_ref[...], 0.0)
    o_ref[...] = dinv_ref[...] * h


def _mid(sp, y1, dinv16, b1t):
    return pl.pallas_call(
        _mid_body,
        out_shape=jax.ShapeDtypeStruct((NPF, 128), jnp.float32),
    )(sp, y1, dinv16, b1t)


def _final_body(sp_ref, y2_ref, dinv_ref, wb_ref, b_ref, o_hbm, of_ref, dsem):
    nf = N * D_HID // 128
    zf = dinv_ref[:nf]
    zf = zf * (sp_ref[0, :nf] + sp_ref[1, :nf] + y2_ref[:nf])
    of_ref[...] = (
        jnp.dot(zf, wb_ref[...], preferred_element_type=jnp.float32) + b_ref[...]
    )
    pltpu.async_copy(of_ref, o_hbm.reshape(N // 8, 8 * D_OUT), dsem).wait()


def _final(sp, y2, dinv16, w2b, b2t):
    return pl.pallas_call(
        _final_body,
        in_specs=[pl.BlockSpec(memory_space=pltpu.MemorySpace.VMEM)] * 5,
        scratch_shapes=[
            pltpu.VMEM((N // 8, 8 * D_OUT), jnp.float32),
            pltpu.SemaphoreType.DMA,
        ],
        out_specs=pl.BlockSpec(memory_space=pltpu.MemorySpace.HBM),
        out_shape=jax.ShapeDtypeStruct((N, D_OUT), jnp.float32),
    )(sp, y2, dinv16, w2b, b2t)


def kernel(data, edge_index, W1, b1, W2, b2):
    f32 = jnp.float32
    zerosf = jnp.zeros((NPF, 128), f32)
    b1t = jnp.tile(b1, 128 // D_HID).reshape(1, 128)
    w1b = (jnp.eye(8, dtype=f32)[:, None, :, None] * W1[None, :, None, :]).reshape(
        8 * D_IN, 8 * D_HID
    )

    degp = _deg(edge_index, zerosf.reshape(NPF * 128))
    dinvf = _scale(degp.reshape(NC, NP // 128, 128))
    dinv16f = jnp.broadcast_to(
        dinvf.reshape(NP, 1), (NP, D_HID)
    ).reshape(NPF, 128)
    y1f = _mm1(data, w1b, dinv16f)
    s1p = _agg(y1f.reshape(NP, D_HID), edge_index, zerosf.reshape(NP, D_HID))
    y2f = _mid(s1p.reshape(NC, NPF, 128), y1f, dinv16f, b1t)
    s2p = _agg(y2f.reshape(NP, D_HID), edge_index, zerosf.reshape(NP, D_HID))
    w2b = (jnp.eye(8, dtype=f32)[:, None, :, None] * W2[None, :, None, :]).reshape(
        8 * D_HID, 8 * D_OUT
    )
    b2t = jnp.tile(b2, 8).reshape(1, 8 * D_OUT)
    return _final(s2p.reshape(NC, NPF, 128), y2f, dinv16f, w2b, b2t)


# speedup vs baseline: 1.2235x; 1.0328x over previous
import functools

import jax
import jax.numpy as jnp
from jax import lax
from jax.experimental import pallas as pl
from jax.experimental.pallas import tpu as pltpu
from jax.experimental.pallas import tpu_sc as plsc

N = 10000
E = 320000
D_IN = 128
D_HID = 16
D_OUT = 128

NC = 2
NS = 16
NW = NC * NS
SUB = 128
NCHUNK = E // SUB
CPW = NCHUNK // NW
TAIL = NCHUNK - CPW * NW
NBUF = 13
NP = 10240
NPF = NP * D_HID // 128
ROWS_PER_SUB = NP // NS
WPS = NP // NS

_sc_mesh = plsc.VectorSubcoreMesh(core_axis_name="c", subcore_axis_name="s")


def _fill_indices(edge_hbm, row, wid, idx_v, isem, with_tail):
    base = wid * CPW * SUB

    def fill(t, carry):
        pltpu.async_copy(
            edge_hbm.at[row, pl.ds(base + t * SUB, SUB)], idx_v.at[t], isem
        )
        return carry

    lax.fori_loop(0, CPW, fill, 0)
    if with_tail:
        @pl.when(wid < TAIL)
        def _():
            pltpu.async_copy(
                edge_hbm.at[row, pl.ds((NW * CPW + wid) * SUB, SUB)],
                idx_v.at[CPW],
                isem,
            )

    def drain(t, carry):
        pltpu.make_async_copy(
            edge_hbm.at[row, pl.ds(base + t * SUB, SUB)], idx_v.at[t], isem
        ).wait()
        return carry

    lax.fori_loop(0, CPW, drain, 0)
    if with_tail:
        @pl.when(wid < TAIL)
        def _():
            pltpu.make_async_copy(
                edge_hbm.at[row, pl.ds((NW * CPW + wid) * SUB, SUB)],
                idx_v.at[CPW],
                isem,
            ).wait()


@functools.partial(
    pl.kernel,
    out_type=jax.ShapeDtypeStruct((NC, NP), jnp.float32),
    mesh=_sc_mesh,
    compiler_params=pltpu.CompilerParams(use_tc_tiling_on_sc=False),
    scratch_types=[
        pltpu.VMEM((CPW + 1, SUB), jnp.int32),
        pltpu.VMEM((SUB,), jnp.float32),
        pltpu.SemaphoreType.DMA,
        pltpu.SemaphoreType.DMA,
        pltpu.VMEM_SHARED((NP,), jnp.float32),
    ],
)
def _deg(edge_hbm, zeros_hbm, out_hbm, dst_v, ones_v, isem, dsem, acc):
    cid = lax.axis_index("c")
    sid = lax.axis_index("s")
    wid = sid * NC + cid
    w0 = sid * WPS
    pltpu.sync_copy(zeros_hbm.at[pl.ds(0, WPS)], acc.at[pl.ds(w0, WPS)])
    _fill_indices(edge_hbm, 1, wid, dst_v, isem, with_tail=True)
    for i in range(SUB // 16):
        ones_v[pl.ds(i * 16, 16)] = jnp.full((16,), 1.0, jnp.float32)
    plsc.subcore_barrier()

    def fire(j, carry):
        pltpu.async_copy(ones_v, acc.at[dst_v.at[j]], dsem, add=True)
        return carry

    lax.fori_loop(0, CPW, fire, 0)

    @pl.when(wid < TAIL)
    def _():
        pltpu.async_copy(ones_v, acc.at[dst_v.at[CPW]], dsem, add=True)

    def drain(j, carry):
        pltpu.make_async_copy(ones_v, acc.at[dst_v.at[j]], dsem).wait()
        return carry

    lax.fori_loop(0, CPW, drain, 0)

    @pl.when(wid < TAIL)
    def _():
        pltpu.make_async_copy(ones_v, acc.at[dst_v.at[CPW]], dsem).wait()

    plsc.subcore_barrier()
    pltpu.sync_copy(acc.at[pl.ds(w0, WPS)], out_hbm.at[cid, pl.ds(w0, WPS)])


@functools.partial(
    pl.kernel,
    out_type=jax.ShapeDtypeStruct((NC, NP, D_HID), jnp.float32),
    mesh=_sc_mesh,
    compiler_params=pltpu.CompilerParams(use_tc_tiling_on_sc=False),
    scratch_types=[
        pltpu.VMEM((CPW + 1, SUB), jnp.int32),
        pltpu.VMEM((CPW + 1, SUB), jnp.int32),
        pltpu.VMEM((NBUF, SUB, D_HID), jnp.float32),
        pltpu.SemaphoreType.DMA,
        pltpu.SemaphoreType.DMA((NBUF,)),
        pltpu.VMEM_SHARED((NP, D_HID), jnp.float32),
    ],
)
def _agg(y_hbm, edge_hbm, zeros_hbm, out_hbm, src_v, dst_v, rows_v, isem, sem, acc):
    cid = lax.axis_index("c")
    sid = lax.axis_index("s")
    wid = sid * NC + cid
    r0 = sid * ROWS_PER_SUB
    pltpu.sync_copy(zeros_hbm.at[pl.ds(r0, ROWS_PER_SUB)], acc.at[pl.ds(r0, ROWS_PER_SUB)])
    _fill_indices(edge_hbm, 0, wid, src_v, isem, with_tail=True)
    _fill_indices(edge_hbm, 1, wid, dst_v, isem, with_tail=True)
    plsc.subcore_barrier()

    for b in range(NBUF):
        pltpu.async_copy(y_hbm.at[src_v.at[b]], rows_v.at[b], sem.at[b])

    def outer(o, carry):
        for b in range(NBUF):
            j = o * NBUF + b
            pltpu.make_async_copy(
                y_hbm.at[src_v.at[j]], rows_v.at[b], sem.at[b]
            ).wait()
            pltpu.sync_copy(rows_v.at[b], acc.at[dst_v.at[j]], add=True)
            nxt = j + NBUF

            @pl.when(nxt < CPW)
            def _():
                pltpu.async_copy(y_hbm.at[src_v.at[nxt]], rows_v.at[b], sem.at[b])

        return carry

    lax.fori_loop(0, CPW // NBUF, outer, 0)

    @pl.when(wid < TAIL)
    def _():
        pltpu.async_copy(y_hbm.at[src_v.at[CPW]], rows_v.at[0], sem.at[0])
        pltpu.make_async_copy(y_hbm.at[src_v.at[CPW]], rows_v.at[0], sem.at[0]).wait()
        pltpu.sync_copy(rows_v.at[0], acc.at[dst_v.at[CPW]], add=True)

    plsc.subcore_barrier()
    pltpu.sync_copy(acc.at[pl.ds(r0, ROWS_PER_SUB)], out_hbm.at[cid, pl.ds(r0, ROWS_PER_SUB)])


def _scale_body(degp_ref, dinv_ref):
    deg = degp_ref[0] + degp_ref[1] + 1.0
    dinv_ref[...] = lax.rsqrt(deg)


def _scale(degp):
    return pl.pallas_call(
        _scale_body,
        out_shape=jax.ShapeDtypeStruct((NP // 128, 128), jnp.float32),
    )(degp)


def _mm1_body(x_hbm, wb_ref, o_ref, xf_ref, dsem):
    pltpu.async_copy(x_hbm.reshape(N // 8, 8 * D_IN), xf_ref, dsem).wait()
    xw = jnp.dot(xf_ref[...], wb_ref[...], preferred_element_type=jnp.float32)
    o_ref[: N // 8] = xw
    o_ref[N // 8 :] = jnp.zeros((NPF - N // 8, 128), jnp.float32)


def _mm1(x, wb):
    return pl.pallas_call(
        _mm1_body,
        in_specs=[
            pl.BlockSpec(memory_space=pltpu.MemorySpace.HBM),
            pl.BlockSpec(memory_space=pltpu.MemorySpace.VMEM),
        ],
        scratch_shapes=[
            pltpu.VMEM((N // 8, 8 * D_IN), jnp.float32),
            pltpu.SemaphoreType.DMA,
        ],
        out_shape=jax.ShapeDtypeStruct((NPF, 128), jnp.float32),
    )(x, wb)


def _y1_body(dinv_ref, xw_ref, o_ref):
    o_ref[...] = dinv_ref[...] * xw_ref[...]


def _y1(dinv16f, xwf):
    return pl.pallas_call(
        _y1_body,
        out_shape=jax.ShapeDtypeStruct((NPF, 128), jnp.float32),
    )(dinv16f, xwf)


def _mid_body(sp_ref, y1_ref, dinv_ref, b_ref, o_ref):
    agg = dinv_ref[...] * (sp_ref[0] + sp_ref[1] + y1_ref[...])
    h = jnp.maximum(agg + b_ref[...], 0.0)
    o_ref[...] = dinv_ref[...] * h


def _mid(sp, y1, dinv16, b1t):
    return pl.pallas_call(
        _mid_body,
        out_shape=jax.ShapeDtypeStruct((NPF, 128), jnp.float32),
    )(sp, y1, dinv16, b1t)


def _final_body(sp_ref, y2_ref, dinv_ref, wb_ref, b_ref, o_hbm, of_ref, dsem):
    nf = N * D_HID // 128
    zf = dinv_ref[:nf]
    zf = zf * (sp_ref[0, :nf] + sp_ref[1, :nf] + y2_ref[:nf])
    of_ref[...] = (
        jnp.dot(zf, wb_ref[...], preferred_element_type=jnp.float32) + b_ref[...]
    )
    pltpu.async_copy(of_ref, o_hbm.reshape(N // 8, 8 * D_OUT), dsem).wait()


def _final(sp, y2, dinv16, w2b, b2t):
    return pl.pallas_call(
        _final_body,
        in_specs=[pl.BlockSpec(memory_space=pltpu.MemorySpace.VMEM)] * 5,
        scratch_shapes=[
            pltpu.VMEM((N // 8, 8 * D_OUT), jnp.float32),
            pltpu.SemaphoreType.DMA,
        ],
        out_specs=pl.BlockSpec(memory_space=pltpu.MemorySpace.HBM),
        out_shape=jax.ShapeDtypeStruct((N, D_OUT), jnp.float32),
    )(sp, y2, dinv16, w2b, b2t)


def kernel(data, edge_index, W1, b1, W2, b2):
    f32 = jnp.float32
    zerosf = jnp.zeros((NPF, 128), f32)
    b1t = jnp.tile(b1, 128 // D_HID).reshape(1, 128)
    w1b = (jnp.eye(8, dtype=f32)[:, None, :, None] * W1[None, :, None, :]).reshape(
        8 * D_IN, 8 * D_HID
    )

    degp = _deg(edge_index, zerosf.reshape(NPF * 128))
    dinvf = _scale(degp.reshape(NC, NP // 128, 128))
    dinv16f = jnp.broadcast_to(
        dinvf.reshape(NP, 1), (NP, D_HID)
    ).reshape(NPF, 128)
    xwf = _mm1(data, w1b)
    y1f = _y1(dinv16f, xwf)
    s1p = _agg(y1f.reshape(NP, D_HID), edge_index, zerosf.reshape(NP, D_HID))
    y2f = _mid(s1p.reshape(NC, NPF, 128), y1f, dinv16f, b1t)
    s2p = _agg(y2f.reshape(NP, D_HID), edge_index, zerosf.reshape(NP, D_HID))
    w2b = (jnp.eye(8, dtype=f32)[:, None, :, None] * W2[None, :, None, :]).reshape(
        8 * D_HID, 8 * D_OUT
    )
    b2t = jnp.tile(b2, 8).reshape(1, 8 * D_OUT)
    return _final(s2p.reshape(NC, NPF, 128), y2f, dinv16f, w2b, b2t)


# speedup vs baseline: 1.2302x; 1.0055x over previous
import functools

import jax
import jax.numpy as jnp
from jax import lax
from jax.experimental import pallas as pl
from jax.experimental.pallas import tpu as pltpu
from jax.experimental.pallas import tpu_sc as plsc

N = 10000
E = 320000
D_IN = 128
D_HID = 16
D_OUT = 128

NC = 2
NS = 16
NW = NC * NS
SUB = 128
NCHUNK = E // SUB
CPW = NCHUNK // NW
TAIL = NCHUNK - CPW * NW
NBUF = 13
NP = 10240
NPF = NP * D_HID // 128
ROWS_PER_SUB = NP // NS
WPS = NP // NS

_sc_mesh = plsc.VectorSubcoreMesh(core_axis_name="c", subcore_axis_name="s")


def _fill_indices(edge_hbm, row, wid, idx_v, isem, with_tail):
    base = wid * CPW * SUB

    def fill(t, carry):
        pltpu.async_copy(
            edge_hbm.at[row, pl.ds(base + t * SUB, SUB)], idx_v.at[t], isem
        )
        return carry

    lax.fori_loop(0, CPW, fill, 0)
    if with_tail:
        @pl.when(wid < TAIL)
        def _():
            pltpu.async_copy(
                edge_hbm.at[row, pl.ds((NW * CPW + wid) * SUB, SUB)],
                idx_v.at[CPW],
                isem,
            )

    def drain(t, carry):
        pltpu.make_async_copy(
            edge_hbm.at[row, pl.ds(base + t * SUB, SUB)], idx_v.at[t], isem
        ).wait()
        return carry

    lax.fori_loop(0, CPW, drain, 0)
    if with_tail:
        @pl.when(wid < TAIL)
        def _():
            pltpu.make_async_copy(
                edge_hbm.at[row, pl.ds((NW * CPW + wid) * SUB, SUB)],
                idx_v.at[CPW],
                isem,
            ).wait()


@functools.partial(
    pl.kernel,
    out_type=jax.ShapeDtypeStruct((NC, NP), jnp.float32),
    mesh=_sc_mesh,
    compiler_params=pltpu.CompilerParams(use_tc_tiling_on_sc=True),
    scratch_types=[
        pltpu.VMEM((CPW + 1, SUB), jnp.int32),
        pltpu.VMEM((SUB,), jnp.float32),
        pltpu.SemaphoreType.DMA,
        pltpu.SemaphoreType.DMA,
        pltpu.VMEM_SHARED((NP,), jnp.float32),
    ],
)
def _deg(edge_hbm, zeros_hbm, out_hbm, dst_v, ones_v, isem, dsem, acc):
    cid = lax.axis_index("c")
    sid = lax.axis_index("s")
    wid = sid * NC + cid
    w0 = sid * WPS
    pltpu.sync_copy(zeros_hbm.at[pl.ds(0, WPS)], acc.at[pl.ds(w0, WPS)])
    _fill_indices(edge_hbm, 1, wid, dst_v, isem, with_tail=True)
    for i in range(SUB // 16):
        ones_v[pl.ds(i * 16, 16)] = jnp.full((16,), 1.0, jnp.float32)
    plsc.subcore_barrier()

    def fire(j, carry):
        pltpu.async_copy(ones_v, acc.at[dst_v.at[j]], dsem, add=True)
        return carry

    lax.fori_loop(0, CPW, fire, 0)

    @pl.when(wid < TAIL)
    def _():
        pltpu.async_copy(ones_v, acc.at[dst_v.at[CPW]], dsem, add=True)

    def drain(j, carry):
        pltpu.make_async_copy(ones_v, acc.at[dst_v.at[j]], dsem).wait()
        return carry

    lax.fori_loop(0, CPW, drain, 0)

    @pl.when(wid < TAIL)
    def _():
        pltpu.make_async_copy(ones_v, acc.at[dst_v.at[CPW]], dsem).wait()

    plsc.subcore_barrier()
    pltpu.sync_copy(acc.at[pl.ds(w0, WPS)], out_hbm.at[cid, pl.ds(w0, WPS)])


@functools.partial(
    pl.kernel,
    out_type=jax.ShapeDtypeStruct((NC, NP, D_HID), jnp.float32),
    mesh=_sc_mesh,
    compiler_params=pltpu.CompilerParams(use_tc_tiling_on_sc=False),
    scratch_types=[
        pltpu.VMEM((CPW + 1, SUB), jnp.int32),
        pltpu.VMEM((CPW + 1, SUB), jnp.int32),
        pltpu.VMEM((NBUF, SUB, D_HID), jnp.float32),
        pltpu.SemaphoreType.DMA,
        pltpu.SemaphoreType.DMA((NBUF,)),
        pltpu.VMEM_SHARED((NP, D_HID), jnp.float32),
    ],
)
def _agg(y_hbm, edge_hbm, zeros_hbm, out_hbm, src_v, dst_v, rows_v, isem, sem, acc):
    cid = lax.axis_index("c")
    sid = lax.axis_index("s")
    wid = sid * NC + cid
    r0 = sid * ROWS_PER_SUB
    pltpu.sync_copy(zeros_hbm.at[pl.ds(r0, ROWS_PER_SUB)], acc.at[pl.ds(r0, ROWS_PER_SUB)])
    _fill_indices(edge_hbm, 0, wid, src_v, isem, with_tail=True)
    _fill_indices(edge_hbm, 1, wid, dst_v, isem, with_tail=True)
    plsc.subcore_barrier()

    for b in range(NBUF):
        pltpu.async_copy(y_hbm.at[src_v.at[b]], rows_v.at[b], sem.at[b])

    def outer(o, carry):
        for b in range(NBUF):
            j = o * NBUF + b
            pltpu.make_async_copy(
                y_hbm.at[src_v.at[j]], rows_v.at[b], sem.at[b]
            ).wait()
            pltpu.sync_copy(rows_v.at[b], acc.at[dst_v.at[j]], add=True)
            nxt = j + NBUF

            @pl.when(nxt < CPW)
            def _():
                pltpu.async_copy(y_hbm.at[src_v.at[nxt]], rows_v.at[b], sem.at[b])

        return carry

    lax.fori_loop(0, CPW // NBUF, outer, 0)

    @pl.when(wid < TAIL)
    def _():
        pltpu.async_copy(y_hbm.at[src_v.at[CPW]], rows_v.at[0], sem.at[0])
        pltpu.make_async_copy(y_hbm.at[src_v.at[CPW]], rows_v.at[0], sem.at[0]).wait()
        pltpu.sync_copy(rows_v.at[0], acc.at[dst_v.at[CPW]], add=True)

    plsc.subcore_barrier()
    pltpu.sync_copy(acc.at[pl.ds(r0, ROWS_PER_SUB)], out_hbm.at[cid, pl.ds(r0, ROWS_PER_SUB)])


def _scale_body(degp_ref, dinv_ref):
    deg = degp_ref[0] + degp_ref[1] + 1.0
    dinv_ref[...] = lax.rsqrt(deg)


def _scale(degp):
    return pl.pallas_call(
        _scale_body,
        out_shape=jax.ShapeDtypeStruct((NP // 128, 128), jnp.float32),
    )(degp)


def _mm1_body(x_hbm, wb_ref, o_ref, xf_ref, dsem):
    pltpu.async_copy(x_hbm.reshape(N // 8, 8 * D_IN), xf_ref, dsem).wait()
    xw = jnp.dot(xf_ref[...], wb_ref[...], preferred_element_type=jnp.float32)
    o_ref[: N // 8] = xw
    o_ref[N // 8 :] = jnp.zeros((NPF - N // 8, 128), jnp.float32)


def _mm1(x, wb):
    return pl.pallas_call(
        _mm1_body,
        in_specs=[
            pl.BlockSpec(memory_space=pltpu.MemorySpace.HBM),
            pl.BlockSpec(memory_space=pltpu.MemorySpace.VMEM),
        ],
        scratch_shapes=[
            pltpu.VMEM((N // 8, 8 * D_IN), jnp.float32),
            pltpu.SemaphoreType.DMA,
        ],
        out_shape=jax.ShapeDtypeStruct((NPF, 128), jnp.float32),
    )(x, wb)


def _y1_body(dinv_ref, xw_ref, o_ref):
    o_ref[...] = dinv_ref[...] * xw_ref[...]


def _y1(dinv16f, xwf):
    return pl.pallas_call(
        _y1_body,
        out_shape=jax.ShapeDtypeStruct((NPF, 128), jnp.float32),
    )(dinv16f, xwf)


def _mid_body(sp_ref, y1_ref, dinv_ref, b_ref, o_ref):
    agg = dinv_ref[...] * (sp_ref[0] + sp_ref[1] + y1_ref[...])
    h = jnp.maximum(agg + b_ref[...], 0.0)
    o_ref[...] = dinv_ref[...] * h


def _mid(sp, y1, dinv16, b1t):
    return pl.pallas_call(
        _mid_body,
        out_shape=jax.ShapeDtypeStruct((NPF, 128), jnp.float32),
    )(sp, y1, dinv16, b1t)


def _final_body(sp_ref, y2_ref, dinv_ref, wb_ref, b_ref, o_hbm, of_ref, dsem):
    nf = N * D_HID // 128
    zf = dinv_ref[:nf]
    zf = zf * (sp_ref[0, :nf] + sp_ref[1, :nf] + y2_ref[:nf])
    of_ref[...] = (
        jnp.dot(zf, wb_ref[...], preferred_element_type=jnp.float32) + b_ref[...]
    )
    pltpu.async_copy(of_ref, o_hbm.reshape(N // 8, 8 * D_OUT), dsem).wait()


def _final(sp, y2, dinv16, w2b, b2t):
    return pl.pallas_call(
        _final_body,
        in_specs=[pl.BlockSpec(memory_space=pltpu.MemorySpace.VMEM)] * 5,
        scratch_shapes=[
            pltpu.VMEM((N // 8, 8 * D_OUT), jnp.float32),
            pltpu.SemaphoreType.DMA,
        ],
        out_specs=pl.BlockSpec(memory_space=pltpu.MemorySpace.HBM),
        out_shape=jax.ShapeDtypeStruct((N, D_OUT), jnp.float32),
    )(sp, y2, dinv16, w2b, b2t)


def kernel(data, edge_index, W1, b1, W2, b2):
    f32 = jnp.float32
    zerosf = jnp.zeros((NPF, 128), f32)
    b1t = jnp.tile(b1, 128 // D_HID).reshape(1, 128)
    w1b = (jnp.eye(8, dtype=f32)[:, None, :, None] * W1[None, :, None, :]).reshape(
        8 * D_IN, 8 * D_HID
    )

    degp = _deg(edge_index, zerosf.reshape(NPF * 128))
    dinvf = _scale(degp.reshape(NC, NP // 128, 128))
    dinv16f = jnp.broadcast_to(
        dinvf.reshape(NP, 1), (NP, D_HID)
    ).reshape(NPF, 128)
    xwf = _mm1(data, w1b)
    y1f = _y1(dinv16f, xwf)
    s1p = _agg(y1f.reshape(NP, D_HID), edge_index, zerosf.reshape(NP, D_HID))
    y2f = _mid(s1p.reshape(NC, NPF, 128), y1f, dinv16f, b1t)
    s2p = _agg(y2f.reshape(NP, D_HID), edge_index, zerosf.reshape(NP, D_HID))
    w2b = (jnp.eye(8, dtype=f32)[:, None, :, None] * W2[None, :, None, :]).reshape(
        8 * D_HID, 8 * D_OUT
    )
    b2t = jnp.tile(b2, 8).reshape(1, 8 * D_OUT)
    return _final(s2p.reshape(NC, NPF, 128), y2f, dinv16f, w2b, b2t)


# speedup vs baseline: 1.2374x; 1.0058x over previous
import functools

import jax
import jax.numpy as jnp
from jax import lax
from jax.experimental import pallas as pl
from jax.experimental.pallas import tpu as pltpu
from jax.experimental.pallas import tpu_sc as plsc

N = 10000
E = 320000
D_IN = 128
D_HID = 16
D_OUT = 128

NC = 2
NS = 16
NW = NC * NS
SUB = 128
NCHUNK = E // SUB
CPW = NCHUNK // NW
TAIL = NCHUNK - CPW * NW
NBUF = 13
NP = 10240
NPF = NP * D_HID // 128
ROWS_PER_SUB = NP // NS
WPS = NP // NS

_sc_mesh = plsc.VectorSubcoreMesh(core_axis_name="c", subcore_axis_name="s")


def _fill_indices(edge_hbm, row, wid, idx_v, isem, with_tail):
    base = wid * CPW * SUB

    def fill(t, carry):
        pltpu.async_copy(
            edge_hbm.at[row, pl.ds(base + t * SUB, SUB)], idx_v.at[t], isem
        )
        return carry

    lax.fori_loop(0, CPW, fill, 0)
    if with_tail:
        @pl.when(wid < TAIL)
        def _():
            pltpu.async_copy(
                edge_hbm.at[row, pl.ds((NW * CPW + wid) * SUB, SUB)],
                idx_v.at[CPW],
                isem,
            )

    def drain(t, carry):
        pltpu.make_async_copy(
            edge_hbm.at[row, pl.ds(base + t * SUB, SUB)], idx_v.at[t], isem
        ).wait()
        return carry

    lax.fori_loop(0, CPW, drain, 0)
    if with_tail:
        @pl.when(wid < TAIL)
        def _():
            pltpu.make_async_copy(
                edge_hbm.at[row, pl.ds((NW * CPW + wid) * SUB, SUB)],
                idx_v.at[CPW],
                isem,
            ).wait()


@functools.partial(
    pl.kernel,
    out_type=jax.ShapeDtypeStruct((NC, NP), jnp.float32),
    mesh=_sc_mesh,
    compiler_params=pltpu.CompilerParams(use_tc_tiling_on_sc=True),
    scratch_types=[
        pltpu.VMEM((CPW + 1, SUB), jnp.int32),
        pltpu.VMEM((SUB,), jnp.float32),
        pltpu.SemaphoreType.DMA,
        pltpu.SemaphoreType.DMA,
        pltpu.VMEM_SHARED((NP,), jnp.float32),
    ],
)
def _deg(edge_hbm, zeros_hbm, out_hbm, dst_v, ones_v, isem, dsem, acc):
    cid = lax.axis_index("c")
    sid = lax.axis_index("s")
    wid = sid * NC + cid
    w0 = sid * WPS
    pltpu.sync_copy(zeros_hbm.at[pl.ds(0, WPS)], acc.at[pl.ds(w0, WPS)])
    _fill_indices(edge_hbm, 1, wid, dst_v, isem, with_tail=True)
    for i in range(SUB // 16):
        ones_v[pl.ds(i * 16, 16)] = jnp.full((16,), 1.0, jnp.float32)
    plsc.subcore_barrier()

    def fire(j, carry):
        pltpu.async_copy(ones_v, acc.at[dst_v.at[j]], dsem, add=True)
        return carry

    lax.fori_loop(0, CPW, fire, 0)

    @pl.when(wid < TAIL)
    def _():
        pltpu.async_copy(ones_v, acc.at[dst_v.at[CPW]], dsem, add=True)

    def drain(j, carry):
        pltpu.make_async_copy(ones_v, acc.at[dst_v.at[j]], dsem).wait()
        return carry

    lax.fori_loop(0, CPW, drain, 0)

    @pl.when(wid < TAIL)
    def _():
        pltpu.make_async_copy(ones_v, acc.at[dst_v.at[CPW]], dsem).wait()

    plsc.subcore_barrier()
    pltpu.sync_copy(acc.at[pl.ds(w0, WPS)], out_hbm.at[cid, pl.ds(w0, WPS)])


@functools.partial(
    pl.kernel,
    out_type=jax.ShapeDtypeStruct((NC, NP, D_HID), jnp.float32),
    mesh=_sc_mesh,
    compiler_params=pltpu.CompilerParams(use_tc_tiling_on_sc=False),
    scratch_types=[
        pltpu.VMEM((CPW + 1, SUB), jnp.int32),
        pltpu.VMEM((CPW + 1, SUB), jnp.int32),
        pltpu.VMEM((NBUF, SUB, D_HID), jnp.float32),
        pltpu.SemaphoreType.DMA,
        pltpu.SemaphoreType.DMA((NBUF,)),
        pltpu.SemaphoreType.DMA((NBUF,)),
        pltpu.VMEM_SHARED((NP, D_HID), jnp.float32),
    ],
)
def _agg(y_hbm, edge_hbm, zeros_hbm, out_hbm, src_v, dst_v, rows_v, isem, sem, ssem, acc):
    cid = lax.axis_index("c")
    sid = lax.axis_index("s")
    wid = sid * NC + cid
    r0 = sid * ROWS_PER_SUB
    pltpu.sync_copy(zeros_hbm.at[pl.ds(r0, ROWS_PER_SUB)], acc.at[pl.ds(r0, ROWS_PER_SUB)])
    _fill_indices(edge_hbm, 0, wid, src_v, isem, with_tail=True)
    _fill_indices(edge_hbm, 1, wid, dst_v, isem, with_tail=True)
    plsc.subcore_barrier()

    for b in range(NBUF):
        pltpu.async_copy(y_hbm.at[src_v.at[b]], rows_v.at[b], sem.at[b])

    def outer(o, carry):
        for b in range(NBUF):
            j = o * NBUF + b
            bp = (b - 1) % NBUF
            pltpu.make_async_copy(
                y_hbm.at[src_v.at[j]], rows_v.at[b], sem.at[b]
            ).wait()
            pltpu.async_copy(rows_v.at[b], acc.at[dst_v.at[j]], ssem.at[b], add=True)
            jp = j - 1

            @pl.when(jp >= 0)
            def _():
                pltpu.make_async_copy(
                    rows_v.at[bp], acc.at[dst_v.at[jp]], ssem.at[bp]
                ).wait()
                nxt = jp + NBUF

                @pl.when(nxt < CPW)
                def _():
                    pltpu.async_copy(y_hbm.at[src_v.at[nxt]], rows_v.at[bp], sem.at[bp])

        return carry

    lax.fori_loop(0, CPW // NBUF, outer, 0)
    pltpu.make_async_copy(
        rows_v.at[NBUF - 1], acc.at[dst_v.at[CPW - 1]], ssem.at[NBUF - 1]
    ).wait()

    @pl.when(wid < TAIL)
    def _():
        pltpu.async_copy(y_hbm.at[src_v.at[CPW]], rows_v.at[0], sem.at[0])
        pltpu.make_async_copy(y_hbm.at[src_v.at[CPW]], rows_v.at[0], sem.at[0]).wait()
        pltpu.sync_copy(rows_v.at[0], acc.at[dst_v.at[CPW]], add=True)

    plsc.subcore_barrier()
    pltpu.sync_copy(acc.at[pl.ds(r0, ROWS_PER_SUB)], out_hbm.at[cid, pl.ds(r0, ROWS_PER_SUB)])


def _scale_body(degp_ref, dinv_ref):
    deg = degp_ref[0] + degp_ref[1] + 1.0
    dinv_ref[...] = lax.rsqrt(deg)


def _scale(degp):
    return pl.pallas_call(
        _scale_body,
        out_shape=jax.ShapeDtypeStruct((NP // 128, 128), jnp.float32),
    )(degp)


def _mm1_body(x_hbm, wb_ref, o_ref, xf_ref, dsem):
    pltpu.async_copy(x_hbm.reshape(N // 8, 8 * D_IN), xf_ref, dsem).wait()
    xw = jnp.dot(xf_ref[...], wb_ref[...], preferred_element_type=jnp.float32)
    o_ref[: N // 8] = xw
    o_ref[N // 8 :] = jnp.zeros((NPF - N // 8, 128), jnp.float32)


def _mm1(x, wb):
    return pl.pallas_call(
        _mm1_body,
        in_specs=[
            pl.BlockSpec(memory_space=pltpu.MemorySpace.HBM),
            pl.BlockSpec(memory_space=pltpu.MemorySpace.VMEM),
        ],
        scratch_shapes=[
            pltpu.VMEM((N // 8, 8 * D_IN), jnp.float32),
            pltpu.SemaphoreType.DMA,
        ],
        out_shape=jax.ShapeDtypeStruct((NPF, 128), jnp.float32),
    )(x, wb)


def _y1_body(dinv_ref, xw_ref, o_ref):
    o_ref[...] = dinv_ref[...] * xw_ref[...]


def _y1(dinv16f, xwf):
    return pl.pallas_call(
        _y1_body,
        out_shape=jax.ShapeDtypeStruct((NPF, 128), jnp.float32),
    )(dinv16f, xwf)


def _mid_body(sp_ref, y1_ref, dinv_ref, b_ref, o_ref):
    agg = dinv_ref[...] * (sp_ref[0] + sp_ref[1] + y1_ref[...])
    h = jnp.maximum(agg + b_ref[...], 0.0)
    o_ref[...] = dinv_ref[...] * h


def _mid(sp, y1, dinv16, b1t):
    return pl.pallas_call(
        _mid_body,
        out_shape=jax.ShapeDtypeStruct((NPF, 128), jnp.float32),
    )(sp, y1, dinv16, b1t)


def _final_body(sp_ref, y2_ref, dinv_ref, wb_ref, b_ref, o_hbm, of_ref, dsem):
    nf = N * D_HID // 128
    zf = dinv_ref[:nf]
    zf = zf * (sp_ref[0, :nf] + sp_ref[1, :nf] + y2_ref[:nf])
    of_ref[...] = (
        jnp.dot(zf, wb_ref[...], preferred_element_type=jnp.float32) + b_ref[...]
    )
    pltpu.async_copy(of_ref, o_hbm.reshape(N // 8, 8 * D_OUT), dsem).wait()


def _final(sp, y2, dinv16, w2b, b2t):
    return pl.pallas_call(
        _final_body,
        in_specs=[pl.BlockSpec(memory_space=pltpu.MemorySpace.VMEM)] * 5,
        scratch_shapes=[
            pltpu.VMEM((N // 8, 8 * D_OUT), jnp.float32),
            pltpu.SemaphoreType.DMA,
        ],
        out_specs=pl.BlockSpec(memory_space=pltpu.MemorySpace.HBM),
        out_shape=jax.ShapeDtypeStruct((N, D_OUT), jnp.float32),
    )(sp, y2, dinv16, w2b, b2t)


def kernel(data, edge_index, W1, b1, W2, b2):
    f32 = jnp.float32
    zerosf = jnp.zeros((NPF, 128), f32)
    b1t = jnp.tile(b1, 128 // D_HID).reshape(1, 128)
    w1b = (jnp.eye(8, dtype=f32)[:, None, :, None] * W1[None, :, None, :]).reshape(
        8 * D_IN, 8 * D_HID
    )

    degp = _deg(edge_index, zerosf.reshape(NPF * 128))
    dinvf = _scale(degp.reshape(NC, NP // 128, 128))
    dinv16f = jnp.broadcast_to(
        dinvf.reshape(NP, 1), (NP, D_HID)
    ).reshape(NPF, 128)
    xwf = _mm1(data, w1b)
    y1f = _y1(dinv16f, xwf)
    s1p = _agg(y1f.reshape(NP, D_HID), edge_index, zerosf.reshape(NP, D_HID))
    y2f = _mid(s1p.reshape(NC, NPF, 128), y1f, dinv16f, b1t)
    s2p = _agg(y2f.reshape(NP, D_HID), edge_index, zerosf.reshape(NP, D_HID))
    w2b = (jnp.eye(8, dtype=f32)[:, None, :, None] * W2[None, :, None, :]).reshape(
        8 * D_HID, 8 * D_OUT
    )
    b2t = jnp.tile(b2, 8).reshape(1, 8 * D_OUT)
    return _final(s2p.reshape(NC, NPF, 128), y2f, dinv16f, w2b, b2t)
